# baseline (speedup 1.0000x reference)
import functools

import jax
import jax.numpy as jnp
from jax.experimental import pallas as pl
from jax.experimental.pallas import tpu as pltpu

ENTITY_SIZE = 256
HP, MAXHP, GENDER, STATUS, BCB, TRAPPED, NEWSW, TOXIC, SLEEP, FAINTED, LEVEL, ACTIVE, SPECIES, ABILITY, ITEM, ITEM_EFFECT = range(16)


def _fuse_body(st_ref, ab_ref, it_ref, mv_ref,
               whp_ref, wlv_ref, wac_ref, woh_ref,
               wsp_ref, wab_ref, wit_ref, wmv_ref,
               bhp_ref, blv_ref, bac_ref, boh_ref,
               bsp_ref, bab_ref, bit_ref, bmv_ref,
               wg_ref, wv_ref,
               f0g, f1g, f2g, f3ig, f3fg, f4mg, f4pg,
               f0v, f1v, f2v, f3iv, f3fv, f4mv, f4pv,
               biasg, biasv):
    w0 = jnp.concatenate(
        [whp_ref[...], wlv_ref[...], wac_ref[...], woh_ref[...]], axis=0)
    b0 = bhp_ref[...] + blv_ref[...] + bac_ref[...] + boh_ref[...]

    def dot(a, b):
        return jnp.dot(a, b, preferred_element_type=jnp.float32)

    for (mref, fg0, fg1, fg2, fg3i, fg3f, fg4m, fg4p, bias) in (
            (wg_ref, f0g, f1g, f2g, f3ig, f3fg, f4mg, f4pg, biasg),
            (wv_ref, f0v, f1v, f2v, f3iv, f3fv, f4mv, f4pv, biasv)):
        m0 = mref[0]
        m1 = mref[1]
        m2 = mref[2]
        m3 = mref[3]
        m4 = mref[4]
        bf = jnp.bfloat16
        fg0[...] = dot(w0, m0).astype(bf)
        fg1[...] = dot(st_ref[0:256, :], dot(wsp_ref[...], m1)).astype(bf)
        fg2[...] = dot(ab_ref[...], dot(wab_ref[...], m2)).astype(bf)
        fg3i[...] = dot(it_ref[...], dot(wit_ref[0:128, :], m3)).astype(bf)
        fg3f[...] = dot(wit_ref[128:144, :], m3).astype(bf)
        fg4m[...] = dot(mv_ref[0:256, :], dot(wmv_ref[0:256, :], m4)).astype(bf)
        fg4p[...] = dot(wmv_ref[256:262, :], m4).astype(bf)
        bias[...] = jnp.concatenate([
            dot(b0, m0),
            dot(bsp_ref[...], m1),
            dot(bab_ref[...], m2),
            dot(bit_ref[...], m3),
            4.0 * dot(bmv_ref[...], m4),
        ], axis=0)


def _fuse_tables(p):
    D = ENTITY_SIZE
    f32 = jnp.float32
    outs = (
        [jax.ShapeDtypeStruct(s, jnp.bfloat16) for s in
         ((50, D), (256, D), (256, D), (256, D), (16, D), (256, D), (6, D))] * 2
        + [jax.ShapeDtypeStruct((5, D), f32)] * 2)
    return pl.pallas_call(
        _fuse_body,
        out_shape=tuple(outs),
    )(p['species_table'], p['ability_table'], p['item_table'], p['move_table'],
      p['W_hp'], p['W_level'], p['W_active'], p['W_onehot'],
      p['W_species'], p['W_ability'], p['W_item'], p['W_moves'],
      p['b_hp'][None, :], p['b_level'][None, :], p['b_active'][None, :],
      p['b_onehot'][None, :],
      p['b_species'][None, :], p['b_ability'][None, :], p['b_item'][None, :],
      p['b_moves'][None, :],
      p['Wg'], p['Wv'])


def _oh(x, n, blk):
    i = jax.lax.broadcasted_iota(jnp.int32, (blk, n), 1)
    return (i == x).astype(jnp.bfloat16)


def _bits(x, nbits, blk):
    i = jax.lax.broadcasted_iota(jnp.int32, (blk, nbits), 1)
    mask = jnp.left_shift(jnp.int32(1), i)
    return (jnp.bitwise_and(x, mask) != 0).astype(jnp.bfloat16)


def _main_body(e_ref,
               f0g, f1g, f2g, f3ig, f3fg, f4mg, f4pg,
               f0v, f1v, f2v, f3iv, f3fv, f4mv, f4pv,
               biasg, biasv,
               out_ref, *, blk):
    e = e_ref[...].reshape(blk, 24)
    hp = e[:, 0:1].astype(jnp.float32)
    maxhp = jnp.maximum(e[:, 1:2].astype(jnp.float32), 1.0)
    ratio = jnp.clip(hp / maxhp, 0.0, 1.0)
    hp_token = (1023.0 * ratio).astype(jnp.int32)

    feat0 = jnp.concatenate([
        _bits(hp_token, 10, blk),
        _bits(e[:, 10:11], 7, blk),
        _oh(e[:, 11:12], 2, blk),
        ratio.astype(jnp.bfloat16),
        _oh(e[:, 2:3], 3, blk),
        _oh(e[:, 3:4], 7, blk),
        _oh(e[:, 4:5], 2, blk),
        _oh(e[:, 5:6], 2, blk),
        _oh(e[:, 6:7], 2, blk),
        _oh(e[:, 7:8], 8, blk),
        _oh(e[:, 8:9], 4, blk),
        _oh(e[:, 9:10], 2, blk),
    ], axis=1)
    oh_sp = _oh(e[:, 12:13], 256, blk)
    oh_ab = _oh(e[:, 13:14], 256, blk)
    oh_it = _oh(e[:, 14:15], 256, blk)
    oh_fx = _oh(e[:, 15:16], 16, blk)
    mh_mv = (_oh(e[:, 16:17], 256, blk) + _oh(e[:, 17:18], 256, blk)
             + _oh(e[:, 18:19], 256, blk) + _oh(e[:, 19:20], 256, blk))
    pp6 = (_bits(e[:, 20:21], 6, blk) + _bits(e[:, 21:22], 6, blk)
           + _bits(e[:, 22:23], 6, blk) + _bits(e[:, 23:24], 6, blk))

    def dot(a, b):
        return jnp.dot(a, b, preferred_element_type=jnp.float32)

    bg = biasg[...]
    bv = biasv[...]
    g = [dot(feat0, f0g[...]) + bg[0:1],
         dot(oh_sp, f1g[...]) + bg[1:2],
         dot(oh_ab, f2g[...]) + bg[2:3],
         dot(oh_it, f3ig[...]) + dot(oh_fx, f3fg[...]) + bg[3:4],
         dot(mh_mv, f4mg[...]) + dot(pp6, f4pg[...]) + bg[4:5]]
    v = [dot(feat0, f0v[...]) + bv[0:1],
         dot(oh_sp, f1v[...]) + bv[1:2],
         dot(oh_ab, f2v[...]) + bv[2:3],
         dot(oh_it, f3iv[...]) + dot(oh_fx, f3fv[...]) + bv[3:4],
         dot(mh_mv, f4mv[...]) + dot(pp6, f4pv[...]) + bv[4:5]]

    m = jnp.maximum(jnp.maximum(jnp.maximum(g[0], g[1]),
                                jnp.maximum(g[2], g[3])), g[4])
    es = [jnp.exp(gi - m) for gi in g]
    denom = es[0] + es[1] + es[2] + es[3] + es[4]
    num = es[0] * v[0] + es[1] * v[1] + es[2] * v[2] + es[3] * v[3] + es[4] * v[4]
    out_ref[...] = (num / denom).reshape(out_ref.shape)


def _encode(e, tables, rows):
    n, m = e.shape[0], e.shape[1]
    blk = rows * m
    return pl.pallas_call(
        functools.partial(_main_body, blk=blk),
        grid=(n // rows,),
        in_specs=[pl.BlockSpec((rows, m, 24), lambda i: (i, 0, 0))]
        + [pl.BlockSpec(t.shape, lambda i: (0, 0)) for t in tables],
        out_specs=pl.BlockSpec((rows, m, ENTITY_SIZE), lambda i: (i, 0, 0)),
        out_shape=jax.ShapeDtypeStruct((n, m, ENTITY_SIZE), jnp.float32),
    )(e, *tables)


def kernel(active_entities, side_entities, params):
    tables = _fuse_tables(params)
    active_embeddings = _encode(active_entities, tables, 1024)
    side_embeddings = _encode(side_entities, tables, 128)
    tok = side_entities[..., SPECIES]
    valid_team_mask = (tok != 0) | (tok != 1)
    return active_embeddings, side_embeddings, valid_team_mask


# speedup vs baseline: 1.3486x; 1.3486x over previous
import functools

import jax
import jax.numpy as jnp
from jax.experimental import pallas as pl
from jax.experimental.pallas import tpu as pltpu

ENTITY_SIZE = 256
HP, MAXHP, GENDER, STATUS, BCB, TRAPPED, NEWSW, TOXIC, SLEEP, FAINTED, LEVEL, ACTIVE, SPECIES, ABILITY, ITEM, ITEM_EFFECT = range(16)


def _fuse_body(st_ref, ab_ref, it_ref, mv_ref,
               whp_ref, wlv_ref, wac_ref, woh_ref,
               wsp_ref, wab_ref, wit_ref, wmv_ref,
               bhp_ref, blv_ref, bac_ref, boh_ref,
               bsp_ref, bab_ref, bit_ref, bmv_ref,
               wg_ref, wv_ref,
               f0g, f1g, f2g, f3ig, f3fg, f4mg, f4pg,
               f0v, f1v, f2v, f3iv, f3fv, f4mv, f4pv,
               biasg, biasv):
    w0 = jnp.concatenate(
        [whp_ref[...], wlv_ref[...], wac_ref[...], woh_ref[...]], axis=0)
    b0 = bhp_ref[...] + blv_ref[...] + bac_ref[...] + boh_ref[...]

    def dot(a, b):
        return jnp.dot(a, b, preferred_element_type=jnp.float32)

    for (mref, fg0, fg1, fg2, fg3i, fg3f, fg4m, fg4p, bias) in (
            (wg_ref, f0g, f1g, f2g, f3ig, f3fg, f4mg, f4pg, biasg),
            (wv_ref, f0v, f1v, f2v, f3iv, f3fv, f4mv, f4pv, biasv)):
        m0 = mref[0]
        m1 = mref[1]
        m2 = mref[2]
        m3 = mref[3]
        m4 = mref[4]
        bf = jnp.bfloat16
        fg0[...] = dot(w0, m0).astype(bf)
        fg1[...] = dot(st_ref[0:256, :], dot(wsp_ref[...], m1)).astype(bf)
        fg2[...] = dot(ab_ref[...], dot(wab_ref[...], m2)).astype(bf)
        fg3i[...] = dot(it_ref[...], dot(wit_ref[0:128, :], m3)).astype(bf)
        fg3f[...] = dot(wit_ref[128:144, :], m3).astype(bf)
        fg4m[...] = dot(mv_ref[0:256, :], dot(wmv_ref[0:256, :], m4)).astype(bf)
        fg4p[...] = dot(wmv_ref[256:262, :], m4).astype(bf)
        bias[...] = jnp.concatenate([
            dot(b0, m0),
            dot(bsp_ref[...], m1),
            dot(bab_ref[...], m2),
            dot(bit_ref[...], m3),
            4.0 * dot(bmv_ref[...], m4),
        ], axis=0)


def _fuse_tables(p):
    D = ENTITY_SIZE
    f32 = jnp.float32
    outs = (
        [jax.ShapeDtypeStruct(s, jnp.bfloat16) for s in
         ((50, D), (256, D), (256, D), (256, D), (16, D), (256, D), (6, D))] * 2
        + [jax.ShapeDtypeStruct((5, D), f32)] * 2)
    return pl.pallas_call(
        _fuse_body,
        out_shape=tuple(outs),
    )(p['species_table'], p['ability_table'], p['item_table'], p['move_table'],
      p['W_hp'], p['W_level'], p['W_active'], p['W_onehot'],
      p['W_species'], p['W_ability'], p['W_item'], p['W_moves'],
      p['b_hp'][None, :], p['b_level'][None, :], p['b_active'][None, :],
      p['b_onehot'][None, :],
      p['b_species'][None, :], p['b_ability'][None, :], p['b_item'][None, :],
      p['b_moves'][None, :],
      p['Wg'], p['Wv'])


def _oh(x, n, blk):
    i = jax.lax.broadcasted_iota(jnp.int32, (blk, n), 1)
    return (i == x).astype(jnp.bfloat16)


def _bits(x, nbits, blk):
    i = jax.lax.broadcasted_iota(jnp.int32, (blk, nbits), 1)
    mask = jnp.left_shift(jnp.int32(1), i)
    return (jnp.bitwise_and(x, mask) != 0).astype(jnp.bfloat16)


def _main_body(e_ref,
               f0g, f1g, f2g, f3ig, f3fg, f4mg, f4pg,
               f0v, f1v, f2v, f3iv, f3fv, f4mv, f4pv,
               biasg, biasv,
               out_ref, *, blk):
    e = e_ref[...]
    hp = e[:, 0:1].astype(jnp.float32)
    maxhp = jnp.maximum(e[:, 1:2].astype(jnp.float32), 1.0)
    ratio = jnp.clip(hp / maxhp, 0.0, 1.0)
    hp_token = (1023.0 * ratio).astype(jnp.int32)

    feat0 = jnp.concatenate([
        _bits(hp_token, 10, blk),
        _bits(e[:, 10:11], 7, blk),
        _oh(e[:, 11:12], 2, blk),
        ratio.astype(jnp.bfloat16),
        _oh(e[:, 2:3], 3, blk),
        _oh(e[:, 3:4], 7, blk),
        _oh(e[:, 4:5], 2, blk),
        _oh(e[:, 5:6], 2, blk),
        _oh(e[:, 6:7], 2, blk),
        _oh(e[:, 7:8], 8, blk),
        _oh(e[:, 8:9], 4, blk),
        _oh(e[:, 9:10], 2, blk),
    ], axis=1)
    oh_sp = _oh(e[:, 12:13], 256, blk)
    oh_ab = _oh(e[:, 13:14], 256, blk)
    oh_it = _oh(e[:, 14:15], 256, blk)
    oh_fx = _oh(e[:, 15:16], 16, blk)
    mh_mv = (_oh(e[:, 16:17], 256, blk) + _oh(e[:, 17:18], 256, blk)
             + _oh(e[:, 18:19], 256, blk) + _oh(e[:, 19:20], 256, blk))
    pp6 = (_bits(e[:, 20:21], 6, blk) + _bits(e[:, 21:22], 6, blk)
           + _bits(e[:, 22:23], 6, blk) + _bits(e[:, 23:24], 6, blk))

    def dot(a, b):
        return jnp.dot(a, b, preferred_element_type=jnp.float32)

    bg = biasg[...]
    bv = biasv[...]
    g = [dot(feat0, f0g[...]) + bg[0:1],
         dot(oh_sp, f1g[...]) + bg[1:2],
         dot(oh_ab, f2g[...]) + bg[2:3],
         dot(oh_it, f3ig[...]) + dot(oh_fx, f3fg[...]) + bg[3:4],
         dot(mh_mv, f4mg[...]) + dot(pp6, f4pg[...]) + bg[4:5]]
    v = [dot(feat0, f0v[...]) + bv[0:1],
         dot(oh_sp, f1v[...]) + bv[1:2],
         dot(oh_ab, f2v[...]) + bv[2:3],
         dot(oh_it, f3iv[...]) + dot(oh_fx, f3fv[...]) + bv[3:4],
         dot(mh_mv, f4mv[...]) + dot(pp6, f4pv[...]) + bv[4:5]]

    m = jnp.maximum(jnp.maximum(jnp.maximum(g[0], g[1]),
                                jnp.maximum(g[2], g[3])), g[4])
    es = [jnp.exp(gi - m) for gi in g]
    denom = es[0] + es[1] + es[2] + es[3] + es[4]
    num = es[0] * v[0] + es[1] * v[1] + es[2] * v[2] + es[3] * v[3] + es[4] * v[4]
    out_ref[...] = num / denom


def _encode(e, tables, blk):
    n = e.shape[0]
    return pl.pallas_call(
        functools.partial(_main_body, blk=blk),
        grid=(n // blk,),
        in_specs=[pl.BlockSpec((blk, 24), lambda i: (i, 0))]
        + [pl.BlockSpec(t.shape, lambda i: (0, 0)) for t in tables],
        out_specs=pl.BlockSpec((blk, ENTITY_SIZE), lambda i: (i, 0)),
        out_shape=jax.ShapeDtypeStruct((n, ENTITY_SIZE), jnp.float32),
    )(e, *tables)


def kernel(active_entities, side_entities, params):
    B = active_entities.shape[0]
    tables = _fuse_tables(params)
    out_a = _encode(active_entities.reshape(-1, 24), tables, 2048)
    out_s = _encode(side_entities.reshape(-1, 24), tables, 2048)
    active_embeddings = out_a.reshape(B, -1, ENTITY_SIZE)
    side_embeddings = out_s.reshape(B, -1, ENTITY_SIZE)
    tok = side_entities[..., SPECIES]
    valid_team_mask = (tok != 0) | (tok != 1)
    return active_embeddings, side_embeddings, valid_team_mask
